# Initial kernel scaffold; baseline (speedup 1.0000x reference)
import jax
import jax.numpy as jnp
from jax.experimental import pallas as pl


def kernel(users, items, user_table, item_table, W1, b1, W2, b2):
    raise NotImplementedError("write your pallas kernel here")


# speedup vs baseline: 2.4104x; 2.4104x over previous
import functools

import jax
import jax.numpy as jnp
from jax import lax
from jax.experimental import pallas as pl
from jax.experimental.pallas import tpu as pltpu
from jax.experimental.pallas import tpu_sc as plsc

B = 16384
EMB = 128
NC = 2
NS = 16
NW = NC * NS
B_PER_W = B // NW
CHUNK = 128
N_CHUNKS = B_PER_W // CHUNK

BM = 2048


def _sc_gather_body(user_table, item_table, users, items, u_out, i_out,
                    idx_v, rows_v, sem):
    c = lax.axis_index("c")
    s = lax.axis_index("s")
    wid = s * NC + c
    base = wid * B_PER_W

    def gather_one(table, idx_hbm, out_hbm):
        for ch in range(N_CHUNKS):
            off = base + ch * CHUNK
            pltpu.sync_copy(idx_hbm.at[pl.ds(off, CHUNK)], idx_v)
            pltpu.async_copy(table.at[idx_v], rows_v, sem).wait()
            pltpu.sync_copy(rows_v, out_hbm.at[pl.ds(off, CHUNK)])

    gather_one(user_table, users, u_out)
    gather_one(item_table, items, i_out)


_sc_gather = functools.partial(
    pl.kernel,
    out_type=(
        jax.ShapeDtypeStruct((B, EMB), jnp.float32),
        jax.ShapeDtypeStruct((B, EMB), jnp.float32),
    ),
    mesh=plsc.VectorSubcoreMesh(core_axis_name="c", subcore_axis_name="s"),
    scratch_types=[
        pltpu.VMEM((CHUNK,), jnp.int32),
        pltpu.VMEM((CHUNK, EMB), jnp.float32),
        pltpu.SemaphoreType.DMA,
    ],
)(_sc_gather_body)


def _mlp_body(u_ref, i_ref, w1a_ref, w1b_ref, b1_ref, w2_ref, b2_ref, o_ref):
    h = jnp.dot(u_ref[...], w1a_ref[...], preferred_element_type=jnp.float32)
    h = h + jnp.dot(i_ref[...], w1b_ref[...], preferred_element_type=jnp.float32)
    h = jnp.maximum(h + b1_ref[...], 0.0)
    o_ref[...] = jnp.sum(h * w2_ref[...], axis=1) + b2_ref[0, 0]


def _mlp(u_emb, i_emb, w1a_t, w1b_t, b1, w2, b2):
    grid = (B // BM,)
    return pl.pallas_call(
        _mlp_body,
        grid=grid,
        in_specs=[
            pl.BlockSpec((BM, EMB), lambda i: (i, 0)),
            pl.BlockSpec((BM, EMB), lambda i: (i, 0)),
            pl.BlockSpec((2 * EMB // 2, EMB), lambda i: (0, 0)),
            pl.BlockSpec((2 * EMB // 2, EMB), lambda i: (0, 0)),
            pl.BlockSpec((1, EMB), lambda i: (0, 0)),
            pl.BlockSpec((1, EMB), lambda i: (0, 0)),
            pl.BlockSpec((1, 1), lambda i: (0, 0)),
        ],
        out_specs=pl.BlockSpec((BM,), lambda i: (i,)),
        out_shape=jax.ShapeDtypeStruct((B,), jnp.float32),
    )(u_emb, i_emb, w1a_t, w1b_t, b1, w2, b2)


def kernel(users, items, user_table, item_table, W1, b1, W2, b2):
    u_emb, i_emb = _sc_gather(user_table, item_table, users, items)
    w1a_t = W1[:, :EMB].T
    w1b_t = W1[:, EMB:].T
    b1r = b1.reshape(1, EMB)
    w2r = W2.reshape(1, EMB)
    b2r = b2.reshape(1, 1)
    return _mlp(u_emb, i_emb, w1a_t, w1b_t, b1r, w2r, b2r)


# speedup vs baseline: 2.8497x; 1.1823x over previous
import functools

import jax
import jax.numpy as jnp
from jax import lax
from jax.experimental import pallas as pl
from jax.experimental.pallas import tpu as pltpu
from jax.experimental.pallas import tpu_sc as plsc

B = 16384
EMB = 128
NC = 2
NS = 16
NW = NC * NS
B_PER_W = B // NW
CHUNK = 128
N_CHUNKS = B_PER_W // CHUNK

BM = 2048


NBUF = 6
N_TOTAL = 2 * N_CHUNKS


def _sc_gather_body(user_table, item_table, users, items, u_out, i_out,
                    idx_u, idx_i, bufs, sem_g, sem_s):
    c = lax.axis_index("c")
    s = lax.axis_index("s")
    wid = s * NC + c
    base = wid * B_PER_W

    pltpu.sync_copy(users.at[pl.ds(base, B_PER_W)], idx_u)
    pltpu.sync_copy(items.at[pl.ds(base, B_PER_W)], idx_i)

    chunks = [(user_table, idx_u, ch, u_out) for ch in range(N_CHUNKS)] + \
             [(item_table, idx_i, ch, i_out) for ch in range(N_CHUNKS)]

    def fire_gather(cn, slot):
        table, idx, ch, _ = chunks[cn]
        return pltpu.async_copy(
            table.at[idx.at[pl.ds(ch * CHUNK, CHUNK)]],
            bufs.at[slot], sem_g.at[slot])

    gathers = [fire_gather(cn, cn % NBUF) for cn in range(min(NBUF, N_TOTAL))]
    gathers += [None] * (N_TOTAL - len(gathers))
    stores = [None] * N_TOTAL

    for cn in range(N_TOTAL):
        slot = cn % NBUF
        _, _, ch, out_hbm = chunks[cn]
        off = base + ch * CHUNK
        gathers[cn].wait()
        stores[cn] = pltpu.async_copy(
            bufs.at[slot], out_hbm.at[pl.ds(off, CHUNK)], sem_s.at[slot])
        nxt = cn + NBUF
        if nxt < N_TOTAL:
            stores[cn].wait()
            stores[cn] = None
            gathers[nxt] = fire_gather(nxt, slot)

    for cn in range(N_TOTAL):
        if stores[cn] is not None:
            stores[cn].wait()


_sc_gather = functools.partial(
    pl.kernel,
    out_type=(
        jax.ShapeDtypeStruct((B, EMB), jnp.float32),
        jax.ShapeDtypeStruct((B, EMB), jnp.float32),
    ),
    mesh=plsc.VectorSubcoreMesh(core_axis_name="c", subcore_axis_name="s"),
    scratch_types=[
        pltpu.VMEM((B_PER_W,), jnp.int32),
        pltpu.VMEM((B_PER_W,), jnp.int32),
        pltpu.VMEM((NBUF, CHUNK, EMB), jnp.float32),
        pltpu.SemaphoreType.DMA((NBUF,)),
        pltpu.SemaphoreType.DMA((NBUF,)),
    ],
)(_sc_gather_body)


def _mlp_body(u_ref, i_ref, w1a_ref, w1b_ref, b1_ref, w2_ref, b2_ref, o_ref):
    h = jnp.dot(u_ref[...], w1a_ref[...], preferred_element_type=jnp.float32)
    h = h + jnp.dot(i_ref[...], w1b_ref[...], preferred_element_type=jnp.float32)
    h = jnp.maximum(h + b1_ref[...], 0.0)
    o_ref[...] = jnp.sum(h * w2_ref[...], axis=1) + b2_ref[0, 0]


def _mlp(u_emb, i_emb, w1a_t, w1b_t, b1, w2, b2):
    grid = (B // BM,)
    return pl.pallas_call(
        _mlp_body,
        grid=grid,
        in_specs=[
            pl.BlockSpec((BM, EMB), lambda i: (i, 0)),
            pl.BlockSpec((BM, EMB), lambda i: (i, 0)),
            pl.BlockSpec((2 * EMB // 2, EMB), lambda i: (0, 0)),
            pl.BlockSpec((2 * EMB // 2, EMB), lambda i: (0, 0)),
            pl.BlockSpec((1, EMB), lambda i: (0, 0)),
            pl.BlockSpec((1, EMB), lambda i: (0, 0)),
            pl.BlockSpec((1, 1), lambda i: (0, 0)),
        ],
        out_specs=pl.BlockSpec((BM,), lambda i: (i,)),
        out_shape=jax.ShapeDtypeStruct((B,), jnp.float32),
    )(u_emb, i_emb, w1a_t, w1b_t, b1, w2, b2)


def kernel(users, items, user_table, item_table, W1, b1, W2, b2):
    u_emb, i_emb = _sc_gather(user_table, item_table, users, items)
    w1a_t = W1[:, :EMB].T
    w1b_t = W1[:, EMB:].T
    b1r = b1.reshape(1, EMB)
    w2r = W2.reshape(1, EMB)
    b2r = b2.reshape(1, 1)
    return _mlp(u_emb, i_emb, w1a_t, w1b_t, b1r, w2r, b2r)


# speedup vs baseline: 3.1597x; 1.1088x over previous
import functools

import jax
import jax.numpy as jnp
from jax import lax
from jax.experimental import pallas as pl
from jax.experimental.pallas import tpu as pltpu
from jax.experimental.pallas import tpu_sc as plsc

B = 16384
EMB = 128
NC = 2
NS = 16
NW = NC * NS
B_PER_W = B // NW
CHUNK = 128
N_CHUNKS = B_PER_W // CHUNK

BM = 2048


NBUF = 6
N_TOTAL = 2 * N_CHUNKS


def _sc_gather_body(user_table, item_table, users, items, u_out, i_out,
                    idx_u, idx_i, bufs, sem_g, sem_s):
    c = lax.axis_index("c")
    s = lax.axis_index("s")
    wid = s * NC + c
    base = wid * B_PER_W

    pltpu.sync_copy(users.at[pl.ds(base, B_PER_W)], idx_u)
    pltpu.sync_copy(items.at[pl.ds(base, B_PER_W)], idx_i)

    chunks = [(user_table, idx_u, ch, u_out) for ch in range(N_CHUNKS)] + \
             [(item_table, idx_i, ch, i_out) for ch in range(N_CHUNKS)]

    def fire_gather(cn, slot):
        table, idx, ch, _ = chunks[cn]
        return pltpu.async_copy(
            table.at[idx.at[pl.ds(ch * CHUNK, CHUNK)]],
            bufs.at[slot], sem_g.at[slot])

    gathers = [fire_gather(cn, cn % NBUF) for cn in range(min(NBUF, N_TOTAL))]
    gathers += [None] * (N_TOTAL - len(gathers))
    stores = [None] * N_TOTAL

    for cn in range(N_TOTAL):
        slot = cn % NBUF
        _, _, ch, out_hbm = chunks[cn]
        off = base + ch * CHUNK
        gathers[cn].wait()
        stores[cn] = pltpu.async_copy(
            bufs.at[slot], out_hbm.at[pl.ds(off, CHUNK)], sem_s.at[slot])
        nxt = cn + NBUF
        if nxt < N_TOTAL:
            stores[cn].wait()
            stores[cn] = None
            gathers[nxt] = fire_gather(nxt, slot)

    for cn in range(N_TOTAL):
        if stores[cn] is not None:
            stores[cn].wait()


_sc_gather = functools.partial(
    pl.kernel,
    out_type=(
        jax.ShapeDtypeStruct((B, EMB), jnp.float32),
        jax.ShapeDtypeStruct((B, EMB), jnp.float32),
    ),
    mesh=plsc.VectorSubcoreMesh(core_axis_name="c", subcore_axis_name="s"),
    scratch_types=[
        pltpu.VMEM((B_PER_W,), jnp.int32),
        pltpu.VMEM((B_PER_W,), jnp.int32),
        pltpu.VMEM((NBUF, CHUNK, EMB), jnp.float32),
        pltpu.SemaphoreType.DMA((NBUF,)),
        pltpu.SemaphoreType.DMA((NBUF,)),
    ],
)(_sc_gather_body)


def _mlp_body(u_ref, i_ref, w1a_ref, w1b_ref, b1_ref, w2_ref, b2_ref, o_ref):
    u = u_ref[...].astype(jnp.bfloat16)
    it = i_ref[...].astype(jnp.bfloat16)
    h = jnp.dot(u, w1a_ref[...], preferred_element_type=jnp.float32)
    h = h + jnp.dot(it, w1b_ref[...], preferred_element_type=jnp.float32)
    h = jnp.maximum(h + b1_ref[...], 0.0)
    o_ref[...] = jnp.sum(h * w2_ref[...], axis=1) + b2_ref[0, 0]


def _mlp(u_emb, i_emb, w1a_t, w1b_t, b1, w2, b2):
    grid = (B // BM,)
    return pl.pallas_call(
        _mlp_body,
        grid=grid,
        in_specs=[
            pl.BlockSpec((BM, EMB), lambda i: (i, 0)),
            pl.BlockSpec((BM, EMB), lambda i: (i, 0)),
            pl.BlockSpec((2 * EMB // 2, EMB), lambda i: (0, 0)),
            pl.BlockSpec((2 * EMB // 2, EMB), lambda i: (0, 0)),
            pl.BlockSpec((1, EMB), lambda i: (0, 0)),
            pl.BlockSpec((1, EMB), lambda i: (0, 0)),
            pl.BlockSpec((1, 1), lambda i: (0, 0)),
        ],
        out_specs=pl.BlockSpec((BM,), lambda i: (i,)),
        out_shape=jax.ShapeDtypeStruct((B,), jnp.float32),
    )(u_emb, i_emb, w1a_t, w1b_t, b1, w2, b2)


def kernel(users, items, user_table, item_table, W1, b1, W2, b2):
    u_emb, i_emb = _sc_gather(user_table, item_table, users, items)
    w1a_t = W1[:, :EMB].T.astype(jnp.bfloat16)
    w1b_t = W1[:, EMB:].T.astype(jnp.bfloat16)
    b1r = b1.reshape(1, EMB)
    w2r = W2.reshape(1, EMB)
    b2r = b2.reshape(1, 1)
    return _mlp(u_emb, i_emb, w1a_t, w1b_t, b1r, w2r, b2r)


# speedup vs baseline: 3.6303x; 1.1489x over previous
import functools

import jax
import jax.numpy as jnp
from jax import lax
from jax.experimental import pallas as pl
from jax.experimental.pallas import tpu as pltpu
from jax.experimental.pallas import tpu_sc as plsc

B = 16384
EMB = 128
NC = 2
NS = 16
NW = NC * NS
B_PER_W = B // NW
CHUNK = 128
N_CHUNKS = B_PER_W // CHUNK

BM = 2048


NBUF = 6
N_TOTAL = 2 * N_CHUNKS


def _sc_gather_body(user_table, item_table, users, items, u_out, i_out,
                    idx_u, idx_i, bufs, sem_g, sem_s):
    c = lax.axis_index("c")
    s = lax.axis_index("s")
    wid = s * NC + c
    base = wid * B_PER_W

    pltpu.sync_copy(users.at[pl.ds(base, B_PER_W)], idx_u)
    pltpu.sync_copy(items.at[pl.ds(base, B_PER_W)], idx_i)

    chunks = [(user_table, idx_u, ch, u_out) for ch in range(N_CHUNKS)] + \
             [(item_table, idx_i, ch, i_out) for ch in range(N_CHUNKS)]

    def fire_gather(cn, slot):
        table, idx, ch, _ = chunks[cn]
        return pltpu.async_copy(
            table.at[idx.at[pl.ds(ch * CHUNK, CHUNK)]],
            bufs.at[slot], sem_g.at[slot])

    gathers = [fire_gather(cn, cn % NBUF) for cn in range(min(NBUF, N_TOTAL))]
    gathers += [None] * (N_TOTAL - len(gathers))
    stores = [None] * N_TOTAL

    for cn in range(N_TOTAL):
        slot = cn % NBUF
        _, _, ch, out_hbm = chunks[cn]
        off = base + ch * CHUNK
        gathers[cn].wait()
        stores[cn] = pltpu.async_copy(
            bufs.at[slot], out_hbm.at[pl.ds(off, CHUNK)], sem_s.at[slot])
        nxt = cn + NBUF
        if nxt < N_TOTAL:
            stores[cn].wait()
            stores[cn] = None
            gathers[nxt] = fire_gather(nxt, slot)

    for cn in range(N_TOTAL):
        if stores[cn] is not None:
            stores[cn].wait()


_sc_gather = functools.partial(
    pl.kernel,
    out_type=(
        jax.ShapeDtypeStruct((B, EMB), jnp.float32),
        jax.ShapeDtypeStruct((B, EMB), jnp.float32),
    ),
    mesh=plsc.VectorSubcoreMesh(core_axis_name="c", subcore_axis_name="s"),
    scratch_types=[
        pltpu.VMEM((B_PER_W,), jnp.int32),
        pltpu.VMEM((B_PER_W,), jnp.int32),
        pltpu.VMEM((NBUF, CHUNK, EMB), jnp.float32),
        pltpu.SemaphoreType.DMA((NBUF,)),
        pltpu.SemaphoreType.DMA((NBUF,)),
    ],
)(_sc_gather_body)


def _mlp_body(u_ref, i_ref, w1a_ref, w1b_ref, b1_ref, w2_ref, b2_ref, o_ref):
    u = u_ref[...].astype(jnp.bfloat16)
    it = i_ref[...].astype(jnp.bfloat16)
    dn = (((1,), (1,)), ((), ()))
    h = lax.dot_general(w1a_ref[...], u, dn, preferred_element_type=jnp.float32)
    h += lax.dot_general(w1b_ref[...], it, dn, preferred_element_type=jnp.float32)
    h = jnp.maximum(h + b1_ref[...], 0.0)
    s = lax.dot_general(w2_ref[...], h, (((1,), (0,)), ((), ())),
                        preferred_element_type=jnp.float32)
    o_ref[...] = s + b2_ref[0, 0]


def _mlp(u_emb, i_emb, w1a, w1b, b1, w2, b2):
    grid = (B // BM,)
    return pl.pallas_call(
        _mlp_body,
        grid=grid,
        in_specs=[
            pl.BlockSpec((BM, EMB), lambda i: (i, 0)),
            pl.BlockSpec((BM, EMB), lambda i: (i, 0)),
            pl.BlockSpec((EMB, EMB), lambda i: (0, 0)),
            pl.BlockSpec((EMB, EMB), lambda i: (0, 0)),
            pl.BlockSpec((EMB, 1), lambda i: (0, 0)),
            pl.BlockSpec((1, EMB), lambda i: (0, 0)),
            pl.BlockSpec((1, 1), lambda i: (0, 0)),
        ],
        out_specs=pl.BlockSpec((1, BM), lambda i: (0, i)),
        out_shape=jax.ShapeDtypeStruct((1, B), jnp.float32),
    )(u_emb, i_emb, w1a, w1b, b1, w2, b2)


def kernel(users, items, user_table, item_table, W1, b1, W2, b2):
    u_emb, i_emb = _sc_gather(user_table, item_table, users, items)
    w1a = W1[:, :EMB].astype(jnp.bfloat16)
    w1b = W1[:, EMB:].astype(jnp.bfloat16)
    b1r = b1.reshape(EMB, 1)
    w2r = W2.reshape(1, EMB)
    b2r = b2.reshape(1, 1)
    return _mlp(u_emb, i_emb, w1a, w1b, b1r, w2r, b2r).reshape(B)
